# baseline (speedup 1.0000x reference)
import functools

import jax
import jax.numpy as jnp
from jax import lax
from jax.experimental import pallas as pl
from jax.experimental.pallas import tpu as pltpu
from jax.experimental.pallas import tpu_sc as plsc

MAX_LEN = 8192
HIDDEN = 2048
BATCH = 4
T_LEN = 4096
B_TOTAL = BATCH * T_LEN

_NC = 2
_NS = 16
_NW = _NC * _NS
_ROWS_PER_W = MAX_LEN // _NW
_CH = 16
_NCHUNK = _ROWS_PER_W // _CH
_NVREG = B_TOTAL // 16


def _make_lookup():
    mesh = plsc.VectorSubcoreMesh(core_axis_name="c", subcore_axis_name="s")

    @functools.partial(
        pl.kernel,
        mesh=mesh,
        compiler_params=pltpu.CompilerParams(needs_layout_passes=False),
        out_type=jax.ShapeDtypeStruct((B_TOTAL, HIDDEN), jnp.float32),
        scratch_types=[
            pltpu.VMEM((B_TOTAL,), jnp.int32),
            pltpu.VMEM((B_TOTAL + 16,), jnp.int32),
            pltpu.VMEM((B_TOTAL + 16,), jnp.int32),
            pltpu.VMEM((2, _CH, HIDDEN), jnp.float32),
            pltpu.SemaphoreType.DMA,
            pltpu.SemaphoreType.DMA,
            pltpu.SemaphoreType.DMA,
            pltpu.SemaphoreType.DMA,
        ],
    )
    def lookup_kernel(x_hbm, table_hbm, out_hbm, idx_all, packed, stage,
                      chunk_buf, rsem0, rsem1, wsem0, wsem1):
        rsems = (rsem0, rsem1)
        wsems = (wsem0, wsem1)
        wid = lax.axis_index("s") * _NC + lax.axis_index("c")
        lo = wid * _ROWS_PER_W
        lane = lax.iota(jnp.int32, 16)

        def slab_src(c):
            return table_hbm.at[pl.ds(wid * _ROWS_PER_W + c * _CH, _CH)]

        for b in range(2):
            pltpu.async_copy(slab_src(b), chunk_buf.at[b], rsems[b])

        for i in range(BATCH):
            pltpu.sync_copy(x_hbm.at[i], idx_all.at[pl.ds(i * T_LEN, T_LEN)])

        def p1_body(v, n):
            vec = idx_all[pl.ds(v * 16, 16)]
            match = (vec >= lo) & (vec < lo + _ROWS_PER_W)
            e = (lane + v * 16) | ((vec - lo) << 14)
            mi = jnp.where(match, jnp.int32(1), jnp.int32(0))
            dst = n + plsc.cumsum(mi) - 1
            plsc.store_scatter(packed, [dst], e, mask=match)
            return n + jnp.sum(mi)

        n = lax.fori_loop(0, _NVREG, p1_body, jnp.int32(0))
        nv = (n + 15) // 16

        def scan_chunk(c):
            def body(v, m):
                e = packed[pl.ds(v * 16, 16)]
                valid = (lane + v * 16) < n
                match = ((e >> 18) == c) & valid
                mi = jnp.where(match, jnp.int32(1), jnp.int32(0))
                dst = m + plsc.cumsum(mi) - 1
                plsc.store_scatter(stage, [dst], e, mask=match)
                return m + jnp.sum(mi)

            return lax.fori_loop(0, nv, body, jnp.int32(0))

        def emit_copies(m, b):
            buf = chunk_buf.at[b]

            def one(i, c2):
                v16 = stage[pl.ds((i // 16) * 16, 16)]
                e = jnp.max(jnp.where(lane == (i % 16), v16, 0))
                p = e & 0x3FFF
                r = (e >> 14) & (_CH - 1)
                pltpu.async_copy(
                    buf.at[pl.ds(r, 1)], out_hbm.at[pl.ds(p, 1)], wsems[b]
                )
                return c2

            lax.fori_loop(0, m, one, 0)

        def drain(m, b):
            def one(i, c2):
                pltpu.make_async_copy(
                    chunk_buf.at[b].at[pl.ds(0, 1)],
                    out_hbm.at[pl.ds(0, 1)],
                    wsems[b],
                ).wait()
                return c2

            lax.fori_loop(0, m, one, 0)

        ms = [None] * _NCHUNK
        for c in range(_NCHUNK):
            b = c % 2
            pltpu.make_async_copy(slab_src(c), chunk_buf.at[b], rsems[b]).wait()
            m = scan_chunk(c)
            emit_copies(m, b)
            ms[c] = m
            if c + 2 < _NCHUNK:
                drain(m, b)
                pltpu.async_copy(slab_src(c + 2), chunk_buf.at[b], rsems[b])

        drain(ms[_NCHUNK - 2], (_NCHUNK - 2) % 2)
        drain(ms[_NCHUNK - 1], (_NCHUNK - 1) % 2)

    return lookup_kernel


_lookup = _make_lookup()


def kernel(x, weight):
    batch_size, t_length = x.shape
    out = _lookup(x.astype(jnp.int32), weight)
    return out.reshape(batch_size, t_length, HIDDEN)


# speedup vs baseline: 1.0707x; 1.0707x over previous
import functools

import jax
import jax.numpy as jnp
from jax import lax
from jax.experimental import pallas as pl
from jax.experimental.pallas import tpu as pltpu
from jax.experimental.pallas import tpu_sc as plsc

MAX_LEN = 8192
HIDDEN = 2048
BATCH = 4
T_LEN = 4096
B_TOTAL = BATCH * T_LEN

_NC = 2
_NS = 16
_NW = _NC * _NS
_BPW = B_TOTAL // _NW
_C = 8
_NB = 4
_NCH = _BPW // _C


def _make_gather():
    mesh = plsc.VectorSubcoreMesh(core_axis_name="c", subcore_axis_name="s")

    @functools.partial(
        pl.kernel,
        mesh=mesh,
        out_type=jax.ShapeDtypeStruct((B_TOTAL, HIDDEN), jnp.float32),
        scratch_types=[
            pltpu.VMEM((_BPW,), jnp.int32),
            pltpu.VMEM((_NB, _C, HIDDEN), jnp.float32),
        ]
        + [pltpu.SemaphoreType.DMA] * (2 * _NB),
    )
    def gather_kernel(x_hbm, table_hbm, out_hbm, idx_v, rows_v, *sems):
        gsems = sems[:_NB]
        ssems = sems[_NB:]
        wid = lax.axis_index("s") * _NC + lax.axis_index("c")
        base = wid * _BPW
        pltpu.sync_copy(
            x_hbm.at[wid >> 3].at[pl.ds((wid & 7) * _BPW, _BPW)], idx_v
        )

        def g_src(g):
            return table_hbm.at[idx_v.at[pl.ds(g * _C, _C)]]

        def o_dst(g):
            return out_hbm.at[pl.ds(base + g * _C, _C)]

        def wait_gather(g, b):
            pltpu.make_async_copy(g_src(g), rows_v.at[b], gsems[b]).wait()

        def wait_out(g, b):
            pltpu.make_async_copy(rows_v.at[b], o_dst(g), ssems[b]).wait()

        for b in range(_NB - 1):
            pltpu.async_copy(g_src(b), rows_v.at[b], gsems[b])

        for b in range(_NB):
            g = b
            h = g + _NB - 1
            if g < _NB - 1:
                wait_gather(g, b)
                pltpu.async_copy(rows_v.at[b], o_dst(g), ssems[b])
            bh = h % _NB
            if h >= _NB:
                wait_out(h - _NB, bh)
            pltpu.async_copy(g_src(h), rows_v.at[bh], gsems[bh])
            if g == _NB - 1:
                wait_gather(g, b)
                pltpu.async_copy(rows_v.at[b], o_dst(g), ssems[b])

        def outer(j, carry):
            for b in range(_NB):
                g = j * _NB + b
                wait_gather(g, b)
                pltpu.async_copy(rows_v.at[b], o_dst(g), ssems[b])
                h = g + _NB - 1
                bh = (b + _NB - 1) % _NB
                wait_out(h - _NB, bh)
                pltpu.async_copy(g_src(h), rows_v.at[bh], gsems[bh])
            return carry

        lax.fori_loop(1, _NCH // _NB - 1, outer, 0)

        for b in range(_NB):
            g = _NCH - _NB + b
            wait_gather(g, b)
            pltpu.async_copy(rows_v.at[b], o_dst(g), ssems[b])
            if b == 0:
                h = _NCH - 1
                bh = h % _NB
                wait_out(h - _NB, bh)
                pltpu.async_copy(g_src(h), rows_v.at[bh], gsems[bh])
        for b in range(_NB):
            g = _NCH - _NB + b
            wait_out(g, b)

    return gather_kernel


_gather = _make_gather()


def kernel(x, weight):
    batch_size, t_length = x.shape
    out = _gather(x.astype(jnp.int32), weight)
    return out.reshape(batch_size, t_length, HIDDEN)
